# Initial kernel scaffold; baseline (speedup 1.0000x reference)
import jax
import jax.numpy as jnp
from jax.experimental import pallas as pl


def kernel(ent_ids, rel_ids, src, dst, cls_idx, ent_table, rel_table, W_ent0, W_rel0, a_src0, a_dst0, a_rel0, W_ent1, W_rel1, a_src1, a_dst1, a_rel1):
    raise NotImplementedError("write your pallas kernel here")


# speedup vs baseline: 14.5952x; 14.5952x over previous
import jax
import jax.numpy as jnp
from jax.experimental import pallas as pl

_N_NODES = 10000
_HEADS = 8
_DH = 16
_HID = 128
_HOPS = 4
_ALPHA = 0.1
_SLOPE = 0.2

_NODE_BLK = 2000
_EDGE_BLK = 8000


def _node_proj_kernel(h_ref, w_ref, as_ref, ad_ref, q_ref, ss_ref, sd_ref):
    q = jnp.dot(h_ref[...], w_ref[...], preferred_element_type=jnp.float32)
    q_ref[...] = q
    ss_ref[...] = jnp.dot(q, as_ref[...], preferred_element_type=jnp.float32)
    sd_ref[...] = jnp.dot(q, ad_ref[...], preferred_element_type=jnp.float32)


def _edge_kernel(relf_ref, wr_ref, ar_ref, ssrc_ref, sdst_ref, er_ref, logit_ref):
    er = jnp.dot(relf_ref[...], wr_ref[...], preferred_element_type=jnp.float32)
    er_ref[...] = er
    e = ssrc_ref[...] + sdst_ref[...] + jnp.dot(
        er, ar_ref[...], preferred_element_type=jnp.float32)
    logit_ref[...] = jnp.where(e >= 0.0, e, _SLOPE * e)


def _msg_kernel(fs_ref, er_ref, aw_ref, out_ref):
    out_ref[...] = (fs_ref[...] + er_ref[...]) * aw_ref[...]


def _combine_kernel(agg_ref, f0_ref, out_ref):
    out_ref[...] = (1.0 - _ALPHA) * agg_ref[...] + _ALPHA * f0_ref[...]


def _node_proj(h, W, As, Ad):
    grid = (_N_NODES // _NODE_BLK,)
    return pl.pallas_call(
        _node_proj_kernel,
        grid=grid,
        in_specs=[
            pl.BlockSpec((_NODE_BLK, _HID), lambda i: (i, 0)),
            pl.BlockSpec((_HID, _HID), lambda i: (0, 0)),
            pl.BlockSpec((_HID, _HEADS), lambda i: (0, 0)),
            pl.BlockSpec((_HID, _HEADS), lambda i: (0, 0)),
        ],
        out_specs=[
            pl.BlockSpec((_NODE_BLK, _HID), lambda i: (i, 0)),
            pl.BlockSpec((_NODE_BLK, _HEADS), lambda i: (i, 0)),
            pl.BlockSpec((_NODE_BLK, _HEADS), lambda i: (i, 0)),
        ],
        out_shape=[
            jax.ShapeDtypeStruct((_N_NODES, _HID), jnp.float32),
            jax.ShapeDtypeStruct((_N_NODES, _HEADS), jnp.float32),
            jax.ShapeDtypeStruct((_N_NODES, _HEADS), jnp.float32),
        ],
    )(h, W, As, Ad)


def _edge_stage(rel_f, W_rel, Ar, ssrc, sdst):
    n_edges = rel_f.shape[0]
    grid = (n_edges // _EDGE_BLK,)
    return pl.pallas_call(
        _edge_kernel,
        grid=grid,
        in_specs=[
            pl.BlockSpec((_EDGE_BLK, _HID), lambda i: (i, 0)),
            pl.BlockSpec((_HID, _HID), lambda i: (0, 0)),
            pl.BlockSpec((_HID, _HEADS), lambda i: (0, 0)),
            pl.BlockSpec((_EDGE_BLK, _HEADS), lambda i: (i, 0)),
            pl.BlockSpec((_EDGE_BLK, _HEADS), lambda i: (i, 0)),
        ],
        out_specs=[
            pl.BlockSpec((_EDGE_BLK, _HID), lambda i: (i, 0)),
            pl.BlockSpec((_EDGE_BLK, _HEADS), lambda i: (i, 0)),
        ],
        out_shape=[
            jax.ShapeDtypeStruct((n_edges, _HID), jnp.float32),
            jax.ShapeDtypeStruct((n_edges, _HEADS), jnp.float32),
        ],
    )(rel_f, W_rel, Ar, ssrc, sdst)


def _msg_stage(fs, er, aw):
    n_edges = fs.shape[0]
    grid = (n_edges // _EDGE_BLK,)
    spec = pl.BlockSpec((_EDGE_BLK, _HID), lambda i: (i, 0))
    return pl.pallas_call(
        _msg_kernel,
        grid=grid,
        in_specs=[spec, spec, spec],
        out_specs=spec,
        out_shape=jax.ShapeDtypeStruct((n_edges, _HID), jnp.float32),
    )(fs, er, aw)


def _combine_stage(agg, f0):
    grid = (_N_NODES // _NODE_BLK,)
    spec = pl.BlockSpec((_NODE_BLK, _HID), lambda i: (i, 0))
    return pl.pallas_call(
        _combine_kernel,
        grid=grid,
        in_specs=[spec, spec],
        out_specs=spec,
        out_shape=jax.ShapeDtypeStruct((_N_NODES, _HID), jnp.float32),
    )(agg, f0)


def _blockdiag(a):
    hid = a.shape[0] * a.shape[1]
    rows = jnp.arange(hid)
    return jnp.zeros((hid, a.shape[0]), jnp.float32).at[
        rows, rows // a.shape[1]].set(a.reshape(-1))


def _layer(h, rel_f, src, dst, W_ent, W_rel, a_s, a_d, a_r):
    As, Ad, Ar = _blockdiag(a_s), _blockdiag(a_d), _blockdiag(a_r)
    q, ss, sd = _node_proj(h, W_ent, As, Ad)
    er, logit = _edge_stage(rel_f, W_rel, Ar, ss[src], sd[dst])
    m = jax.ops.segment_max(logit, dst, num_segments=_N_NODES)
    m = jnp.where(jnp.isfinite(m), m, 0.0)
    ex = jnp.exp(logit - m[dst])
    denom = jax.ops.segment_sum(ex, dst, num_segments=_N_NODES)
    attn = ex / (denom[dst] + 1e-9)
    aw = jnp.repeat(attn, _DH, axis=1)
    feat = q
    for _ in range(_HOPS):
        msg = _msg_stage(feat[src], er, aw)
        agg = jax.ops.segment_sum(msg, dst, num_segments=_N_NODES)
        feat = _combine_stage(agg, q)
    if h.shape[1] == _HID:
        feat = feat + h
    return feat


def kernel(ent_ids, rel_ids, src, dst, cls_idx, ent_table, rel_table,
           W_ent0, W_rel0, a_src0, a_dst0, a_rel0,
           W_ent1, W_rel1, a_src1, a_dst1, a_rel1):
    h = jnp.take(ent_table, ent_ids, axis=0)
    rel_f = jnp.take(rel_table, rel_ids, axis=0)
    h = _layer(h, rel_f, src, dst, W_ent0, W_rel0, a_src0, a_dst0, a_rel0)
    h = _layer(h, rel_f, src, dst, W_ent1, W_rel1, a_src1, a_dst1, a_rel1)
    return jnp.take(h, cls_idx, axis=0)
